# Initial kernel scaffold; baseline (speedup 1.0000x reference)
import jax
import jax.numpy as jnp
from jax.experimental import pallas as pl


def kernel(emb, edge_index):
    raise NotImplementedError("write your pallas kernel here")


# speedup vs baseline: 3.2501x; 3.2501x over previous
import functools

import jax
import jax.numpy as jnp
from jax import lax
from jax.experimental import pallas as pl
from jax.experimental.pallas import tpu as pltpu
from jax.experimental.pallas import tpu_sc as plsc

E = 320000
D = 128
NC, NS = 2, 16
NW = NC * NS
EPW = E // NW
C = 80
NCHUNK = EPW // C


def _dot_chunk(rows1_v, rows2_v, out_v):
    lane15 = lax.iota(jnp.int32, 16) == 15

    def body(e, _):
        acc = rows1_v[e, pl.ds(0, 16)] * rows2_v[e, pl.ds(0, 16)]
        for j in range(1, D // 16):
            acc = acc + rows1_v[e, pl.ds(16 * j, 16)] * rows2_v[e, pl.ds(16 * j, 16)]
        csum = plsc.cumsum(acc)
        plsc.store_scatter(
            out_v, [jnp.full((16,), e, jnp.int32)], csum, mask=lane15
        )
        return 0

    lax.fori_loop(0, C, body, 0, unroll=2)


def kernel(emb, edge_index):
    src = edge_index[0].astype(jnp.int32)
    dst = edge_index[1].astype(jnp.int32)

    mesh = plsc.VectorSubcoreMesh(core_axis_name="c", subcore_axis_name="s")

    @functools.partial(
        pl.kernel,
        mesh=mesh,
        out_type=jax.ShapeDtypeStruct((E,), jnp.float32),
        compiler_params=pltpu.CompilerParams(needs_layout_passes=False),
        scratch_types=[
            pltpu.VMEM((C,), jnp.int32),
            pltpu.VMEM((C,), jnp.int32),
            pltpu.VMEM((C, D), jnp.float32),
            pltpu.VMEM((C, D), jnp.float32),
            pltpu.VMEM((C,), jnp.float32),
            pltpu.SemaphoreType.DMA,
            pltpu.SemaphoreType.DMA,
        ],
    )
    def _k(emb_hbm, src_hbm, dst_hbm, out_hbm,
           idx1_v, idx2_v, rows1_v, rows2_v, out_v, sem1, sem2):
        wid = lax.axis_index("s") * NC + lax.axis_index("c")
        wbase = wid * EPW

        def chunk(i, _):
            base = wbase + i * C
            pltpu.sync_copy(src_hbm.at[pl.ds(base, C)], idx1_v)
            pltpu.sync_copy(dst_hbm.at[pl.ds(base, C)], idx2_v)
            cp1 = pltpu.async_copy(emb_hbm.at[idx1_v], rows1_v, sem1)
            cp2 = pltpu.async_copy(emb_hbm.at[idx2_v], rows2_v, sem2)
            cp1.wait()
            cp2.wait()
            _dot_chunk(rows1_v, rows2_v, out_v)
            pltpu.sync_copy(out_v, out_hbm.at[pl.ds(base, C)])
            return 0

        lax.fori_loop(0, NCHUNK, chunk, 0)

    return _k(emb, src, dst)


# speedup vs baseline: 6.8809x; 2.1172x over previous
import functools

import jax
import jax.numpy as jnp
from jax import lax
from jax.experimental import pallas as pl
from jax.experimental.pallas import tpu as pltpu
from jax.experimental.pallas import tpu_sc as plsc

E = 320000
D = 128
NC, NS = 2, 16
NW = NC * NS
EPW = E // NW
C = 80
NCHUNK = EPW // C


def _dot_chunk(rows1_v, rows2_v, out_v, obase):
    lane15 = lax.iota(jnp.int32, 16) == 15

    def body(e, _):
        acc = rows1_v[e, pl.ds(0, 16)] * rows2_v[e, pl.ds(0, 16)]
        for j in range(1, D // 16):
            acc = acc + rows1_v[e, pl.ds(16 * j, 16)] * rows2_v[e, pl.ds(16 * j, 16)]
        csum = plsc.cumsum(acc)
        plsc.store_scatter(
            out_v, [jnp.full((16,), obase + e, jnp.int32)], csum, mask=lane15
        )
        return 0

    lax.fori_loop(0, C, body, 0, unroll=2)


def kernel(emb, edge_index):
    src = edge_index[0].astype(jnp.int32)
    dst = edge_index[1].astype(jnp.int32)

    mesh = plsc.VectorSubcoreMesh(core_axis_name="c", subcore_axis_name="s")

    @functools.partial(
        pl.kernel,
        mesh=mesh,
        out_type=jax.ShapeDtypeStruct((E,), jnp.float32),
        compiler_params=pltpu.CompilerParams(needs_layout_passes=False),
        scratch_types=[
            pltpu.VMEM((EPW,), jnp.int32),
            pltpu.VMEM((EPW,), jnp.int32),
            pltpu.VMEM((C, D), jnp.float32),
            pltpu.VMEM((C, D), jnp.float32),
            pltpu.VMEM((C, D), jnp.float32),
            pltpu.VMEM((C, D), jnp.float32),
            pltpu.VMEM((EPW,), jnp.float32),
            pltpu.SemaphoreType.DMA,
            pltpu.SemaphoreType.DMA,
            pltpu.SemaphoreType.DMA,
            pltpu.SemaphoreType.DMA,
        ],
    )
    def _k(emb_hbm, src_hbm, dst_hbm, out_hbm,
           idx1_all, idx2_all, r1a, r1b, r2a, r2b, out_all,
           s1a, s1b, s2a, s2b):
        wid = lax.axis_index("s") * NC + lax.axis_index("c")
        wbase = wid * EPW

        pltpu.sync_copy(src_hbm.at[pl.ds(wbase, EPW)], idx1_all)
        pltpu.sync_copy(dst_hbm.at[pl.ds(wbase, EPW)], idx2_all)

        def fire(i, r1, r2, s1, s2):
            off = pl.ds(i * C, C)
            pltpu.async_copy(emb_hbm.at[idx1_all.at[off]], r1, s1)
            pltpu.async_copy(emb_hbm.at[idx2_all.at[off]], r2, s2)

        def wait(r1, r2, s1, s2):
            off = pl.ds(0, C)
            pltpu.make_async_copy(emb_hbm.at[idx1_all.at[off]], r1, s1).wait()
            pltpu.make_async_copy(emb_hbm.at[idx2_all.at[off]], r2, s2).wait()

        fire(0, r1a, r2a, s1a, s2a)

        def body(k, _):
            i0 = 2 * k
            wait(r1a, r2a, s1a, s2a)
            fire(i0 + 1, r1b, r2b, s1b, s2b)
            _dot_chunk(r1a, r2a, out_all, i0 * C)
            wait(r1b, r2b, s1b, s2b)
            fire(i0 + 2, r1a, r2a, s1a, s2a)
            _dot_chunk(r1b, r2b, out_all, (i0 + 1) * C)
            return 0

        lax.fori_loop(0, (NCHUNK - 1) // 2, body, 0)

        wait(r1a, r2a, s1a, s2a)
        _dot_chunk(r1a, r2a, out_all, (NCHUNK - 1) * C)
        pltpu.sync_copy(out_all, out_hbm.at[pl.ds(wbase, EPW)])

    return _k(emb, src, dst)
